# baseline (speedup 1.0000x reference)
import functools

import jax
import jax.numpy as jnp
from jax import lax
from jax.experimental import pallas as pl
from jax.experimental.pallas import tpu as pltpu
from jax.experimental.pallas import tpu_sc as plsc

N_NODES = 10000
NP = 10240
D = 16
F = 128
E = 320000
NW = 32
CH = 128
NCH = 81
KG = 9
EW = NCH * CH
EP = NW * EW
EC = E // CH
LC = (EP - E) // CH
B30 = EC - 30 * NCH
SN = N_NODES // 16

_mesh = plsc.VectorSubcoreMesh(core_axis_name="c", subcore_axis_name="s")
_sc_params = pltpu.CompilerParams(use_tc_tiling_on_sc=False,
                                  skip_device_barrier=True)
_tc_params = pltpu.CompilerParams(skip_device_barrier=True)


def _rsqrt16(d):
    i = lax.bitcast_convert_type(d, jnp.int32)
    y = lax.bitcast_convert_type(0x5F3759DF - jnp.right_shift(i, 1),
                                 jnp.float32)
    for _ in range(3):
        y = y * (1.5 - 0.5 * d * y * y)
    return y


def _load_idx(tbl_hbm, loops_hbm, base, buf, wid):
    @pl.when(wid <= 29)
    def _all_edges():
        pltpu.sync_copy(tbl_hbm.at[pl.ds(base + wid * NCH, NCH)], buf)

    @pl.when(wid == 30)
    def _boundary():
        pltpu.sync_copy(tbl_hbm.at[pl.ds(base + 30 * NCH, B30)],
                        buf.at[pl.ds(0, B30)])
        pltpu.sync_copy(loops_hbm.at[pl.ds(0, NCH - B30)],
                        buf.at[pl.ds(B30, NCH - B30)])

    @pl.when(wid == 31)
    def _all_loops():
        pltpu.sync_copy(loops_hbm.at[pl.ds(NCH - B30, NCH)], buf)


def _unpack_idx(sv, dv):
    def ub(r, carry):
        for k in range(CH // 16):
            v = sv[r, pl.ds(k * 16, 16)]
            dv[r, pl.ds(k * 16, 16)] = jnp.bitwise_and(v, 16383)
            sv[r, pl.ds(k * 16, 16)] = jnp.right_shift(v, 14)
        return carry
    lax.fori_loop(0, NCH, ub, 0)


@functools.partial(
    pl.kernel,
    out_type=jax.ShapeDtypeStruct((2, 2, N_NODES, D), jnp.float32),
    mesh=_mesh,
    scratch_types=[
        pltpu.VMEM((NCH, CH), jnp.int32),
        pltpu.VMEM((NCH, CH), jnp.int32),
        pltpu.VMEM((CH, D), jnp.float32),
        pltpu.VMEM((NP // 16, D), jnp.float32),
        pltpu.SemaphoreType.DMA,
        pltpu.VMEM_SHARED((NP, D), jnp.float32),
    ],
    compiler_params=_sc_params,
)
def _deg_kernel(edges_hbm, loops_hbm, ones_hbm, out_hbm,
                dpv, dnv, ones_v, zbuf, sem_s, acc):
    cid = lax.axis_index("c")
    sid = lax.axis_index("s")
    wid = cid * 16 + sid
    st = NP // 16
    stg = N_NODES // 16
    zrow = jnp.zeros((16,), jnp.float32)

    def zero_acc():
        def zb(r, carry):
            zbuf[r] = zrow
            return carry
        lax.fori_loop(0, st, zb, 0)
        pltpu.sync_copy(zbuf, acc.at[pl.ds(sid * st, st)])

    zero_acc()
    pltpu.sync_copy(ones_hbm, ones_v)
    _load_idx(edges_hbm, loops_hbm, 0, dpv, wid)
    _load_idx(edges_hbm, loops_hbm, EC, dnv, wid)

    def mask_dst(dv):
        def ub(r, carry):
            for k in range(CH // 16):
                dv[r, pl.ds(k * 16, 16)] = jnp.bitwise_and(
                    dv[r, pl.ds(k * 16, 16)], 16383)
            return carry
        lax.fori_loop(0, NCH, ub, 0)

    mask_dst(dpv)
    mask_dst(dnv)
    plsc.subcore_barrier()

    def one_sign(dv, acc):
        def fire(j, carry):
            pltpu.async_copy(ones_v, acc.at[dv.at[j]], sem_s, add=True)
            return carry
        lax.fori_loop(0, NCH, fire, 0)

        def drain(j, carry):
            pltpu.make_async_copy(ones_v, acc.at[dv.at[0]], sem_s).wait()
            return carry
        lax.fori_loop(0, NCH, drain, 0)

    one_sign(dpv, acc)
    plsc.subcore_barrier()
    pltpu.sync_copy(acc.at[pl.ds(sid * stg, stg)],
                    out_hbm.at[cid, 0, pl.ds(sid * stg, stg)])
    plsc.subcore_barrier()
    zero_acc()
    plsc.subcore_barrier()
    one_sign(dnv, acc)
    plsc.subcore_barrier()
    pltpu.sync_copy(acc.at[pl.ds(sid * stg, stg)],
                    out_hbm.at[cid, 1, pl.ds(sid * stg, stg)])


@functools.partial(
    pl.kernel,
    out_type=jax.ShapeDtypeStruct((4 * N_NODES, D), jnp.float32),
    mesh=_mesh,
    scratch_types=[
        pltpu.VMEM((NCH, CH), jnp.int32),
        pltpu.VMEM((NCH, CH), jnp.int32),
        pltpu.VMEM((2, KG, CH, D), jnp.float32),
        pltpu.VMEM((NP // 16, D), jnp.float32),
        pltpu.VMEM((NP // 16, D), jnp.float32),
        pltpu.VMEM((SN, D), jnp.float32),
        pltpu.VMEM((SN, D), jnp.float32),
        pltpu.SemaphoreType.DMA,
        pltpu.SemaphoreType.DMA,
        pltpu.VMEM_SHARED((NP, D), jnp.float32),
        pltpu.VMEM_SHARED((N_NODES, D), jnp.float32),
    ],
    compiler_params=_sc_params,
)
def _conv_kernel(h_hbm, deg_hbm, edges_hbm, loops_hbm,
                 y_hbm, spv, dpv, rows, bufa, bufb,
                 disp_v, disn_v, sem_g, sem_s, acc, gs):
    cid = lax.axis_index("c")
    sid = lax.axis_index("s")
    wid = cid * 16 + sid
    st = NP // 16
    zrow = jnp.zeros((16,), jnp.float32)
    stripe = pl.ds(sid * SN, SN)

    def zero_acc():
        def zbody(r, carry):
            bufb[r] = zrow
            return carry
        lax.fori_loop(0, st, zbody, 0)
        pltpu.sync_copy(bufb, acc.at[pl.ds(sid * st, st)])

    def prep(sign, dis_v):
        pltpu.sync_copy(
            deg_hbm.at[pl.ds(sign * N_NODES + sid * SN, SN)],
            bufa.at[pl.ds(0, SN)])
        pltpu.sync_copy(
            deg_hbm.at[pl.ds((2 + sign) * N_NODES + sid * SN, SN)],
            bufb.at[pl.ds(0, SN)])

        def dbody(r, carry):
            dis_v[r] = _rsqrt16(bufa[r] + bufb[r])
            return carry
        lax.fori_loop(0, SN, dbody, 0)
        pltpu.sync_copy(h_hbm.at[pl.ds(sign * N_NODES + sid * SN, SN)],
                        bufa.at[pl.ds(0, SN)])

        def gbody(r, carry):
            bufb[r] = bufa[r] * dis_v[r]
            return carry
        lax.fori_loop(0, SN, gbody, 0)
        pltpu.sync_copy(bufb.at[pl.ds(0, SN)], gs.at[stripe])

    zero_acc()
    prep(0, disp_v)
    _load_idx(edges_hbm, loops_hbm, 0, spv, wid)
    _unpack_idx(spv, dpv)
    plsc.subcore_barrier()

    NG = NCH // KG

    def one_sign(g_hbm, sv, dv, acc):
        for k in range(KG):
            pltpu.async_copy(g_hbm.at[sv.at[k]], rows.at[0, k], sem_g)

        def group(t, carry):
            par = lax.rem(t, 2)
            nxt = 1 - par
            base = t * KG

            @pl.when(t + 1 < NG)
            def _fire_next():
                @pl.when(t >= 1)
                def _drain_prev_scatters():
                    for k in range(KG):
                        pltpu.make_async_copy(
                            rows.at[nxt, k],
                            acc.at[dv.at[base - KG + k]], sem_s).wait()
                for k in range(KG):
                    pltpu.async_copy(g_hbm.at[sv.at[base + KG + k]],
                                     rows.at[nxt, k], sem_g)

            for k in range(KG):
                pltpu.make_async_copy(g_hbm.at[sv.at[base + k]],
                                      rows.at[par, k], sem_g).wait()
                pltpu.async_copy(rows.at[par, k], acc.at[dv.at[base + k]],
                                 sem_s, add=True)
            return carry

        lax.fori_loop(0, NG, group, 0)
        for k in range(2 * KG):
            pltpu.make_async_copy(rows.at[0, 0], acc.at[dv.at[0]],
                                  sem_s).wait()

    def post(sign, dis_v):
        pltpu.sync_copy(acc.at[stripe], bufa.at[pl.ds(0, SN)])

        def sbody(r, carry):
            bufb[r] = bufa[r] * dis_v[r]
            return carry
        lax.fori_loop(0, SN, sbody, 0)
        pltpu.sync_copy(
            bufb.at[pl.ds(0, SN)],
            y_hbm.at[pl.ds((sign * 2 + cid) * N_NODES + sid * SN, SN)])

    one_sign(gs, spv, dpv, acc)
    plsc.subcore_barrier()
    post(0, disp_v)
    plsc.subcore_barrier()
    zero_acc()
    prep(1, disn_v)
    _load_idx(edges_hbm, loops_hbm, EC, spv, wid)
    _unpack_idx(spv, dpv)
    plsc.subcore_barrier()
    one_sign(gs, spv, dpv, acc)
    plsc.subcore_barrier()
    post(1, disn_v)


_GRID = 5
_BR = N_NODES // _GRID


def _mm1_body(x_ref, w1p_ref, w1n_ref, h_ref):
    h_ref[0] = jnp.dot(x_ref[...], w1p_ref[...],
                       preferred_element_type=jnp.float32)
    h_ref[1] = jnp.dot(x_ref[...], w1n_ref[...],
                       preferred_element_type=jnp.float32)


_mm1 = pl.pallas_call(
    _mm1_body,
    grid=(_GRID,),
    in_specs=[
        pl.BlockSpec((_BR, F), lambda i: (i, 0)),
        pl.BlockSpec((F, D), lambda i: (0, 0)),
        pl.BlockSpec((F, D), lambda i: (0, 0)),
    ],
    out_specs=pl.BlockSpec((2, _BR, D), lambda i: (0, i, 0)),
    out_shape=jax.ShapeDtypeStruct((2, N_NODES, D), jnp.float32),
    compiler_params=_tc_params,
)


def _mm2_body(ypp_ref, ynp_ref, b1p_ref, b1n_ref, w2p_ref, w2n_ref, h2_ref):
    ap = jnp.maximum(ypp_ref[0] + ypp_ref[1] + b1p_ref[...], 0.0)
    an = jnp.maximum(ynp_ref[0] + ynp_ref[1] + b1n_ref[...], 0.0)
    h = ap - an
    h2_ref[0] = jnp.dot(h, w2p_ref[...], preferred_element_type=jnp.float32)
    h2_ref[1] = jnp.dot(h, w2n_ref[...], preferred_element_type=jnp.float32)


_mm2 = pl.pallas_call(
    _mm2_body,
    grid=(_GRID,),
    in_specs=[
        pl.BlockSpec((2, _BR, D), lambda i: (0, i, 0)),
        pl.BlockSpec((2, _BR, D), lambda i: (0, i, 0)),
        pl.BlockSpec((1, D), lambda i: (0, 0)),
        pl.BlockSpec((1, D), lambda i: (0, 0)),
        pl.BlockSpec((D, D), lambda i: (0, 0)),
        pl.BlockSpec((D, D), lambda i: (0, 0)),
    ],
    out_specs=pl.BlockSpec((2, _BR, D), lambda i: (0, i, 0)),
    out_shape=jax.ShapeDtypeStruct((2, N_NODES, D), jnp.float32),
    compiler_params=_tc_params,
)


def _fin_body(ypp_ref, ynp_ref, b2p_ref, b2n_ref, out_ref):
    op = jnp.maximum(ypp_ref[0] + ypp_ref[1] + b2p_ref[...], 0.0)
    on = jnp.maximum(ynp_ref[0] + ynp_ref[1] + b2n_ref[...], 0.0)
    o = op - on
    m = jnp.max(o, axis=1, keepdims=True)
    lse = jnp.log(jnp.sum(jnp.exp(o - m), axis=1, keepdims=True)) + m
    out_ref[...] = o - lse


_fin = pl.pallas_call(
    _fin_body,
    grid=(_GRID,),
    in_specs=[
        pl.BlockSpec((2, _BR, D), lambda i: (0, i, 0)),
        pl.BlockSpec((2, _BR, D), lambda i: (0, i, 0)),
        pl.BlockSpec((1, D), lambda i: (0, 0)),
        pl.BlockSpec((1, D), lambda i: (0, 0)),
    ],
    out_specs=pl.BlockSpec((_BR, D), lambda i: (i, 0)),
    out_shape=jax.ShapeDtypeStruct((N_NODES, D), jnp.float32),
    compiler_params=_tc_params,
)


def kernel(x, edge_index_pos, edge_index_neg,
           W1p, b1p, W1n, b1n, W2p, b2p, W2n, b2n):
    eip32 = edge_index_pos.astype(jnp.int32)
    ein32 = edge_index_neg.astype(jnp.int32)
    edges_r = jnp.stack([eip32[0] * 16384 + eip32[1],
                         ein32[0] * 16384 + ein32[1]]).reshape(2 * EC, CH)
    loop = jnp.arange(N_NODES, dtype=jnp.int32)
    loops_r = jnp.concatenate(
        [loop * 16385, jnp.full((LC * CH - N_NODES,), NP - 1, jnp.int32)]
    ).reshape(LC, CH)
    ones_tbl = jnp.ones((CH, D), jnp.float32)

    deg = _deg_kernel(edges_r, loops_r, ones_tbl)
    h1 = _mm1(x, W1p, W1n)
    y1 = _conv_kernel(h1.reshape(2 * N_NODES, D),
                      deg.reshape(4 * N_NODES, D),
                      edges_r, loops_r).reshape(2, 2, N_NODES, D)
    h2 = _mm2(y1[0], y1[1], b1p.reshape(1, D), b1n.reshape(1, D), W2p, W2n)
    y2 = _conv_kernel(h2.reshape(2 * N_NODES, D),
                      deg.reshape(4 * N_NODES, D),
                      edges_r, loops_r).reshape(2, 2, N_NODES, D)
    return _fin(y2[0], y2[1], b2p.reshape(1, D), b2n.reshape(1, D))


# speedup vs baseline: 1.2541x; 1.2541x over previous
import functools

import jax
import jax.numpy as jnp
from jax import lax
from jax.experimental import pallas as pl
from jax.experimental.pallas import tpu as pltpu
from jax.experimental.pallas import tpu_sc as plsc

N_NODES = 10000
NP = 10240
D = 16
F = 128
E = 320000
NW = 32
CH = 128
NCH = 81
KG = 9
EW = NCH * CH
EP = NW * EW
EC = E // CH
LC = (EP - E) // CH
B30 = EC - 30 * NCH

_mesh = plsc.VectorSubcoreMesh(core_axis_name="c", subcore_axis_name="s")
_sc_params = pltpu.CompilerParams(use_tc_tiling_on_sc=False,
                                  skip_device_barrier=True)
_tc_params = pltpu.CompilerParams(skip_device_barrier=True)


def _load_idx(tbl_hbm, loops_hbm, row, buf, wid):
    @pl.when(wid <= 29)
    def _all_edges():
        pltpu.sync_copy(tbl_hbm.at[row, pl.ds(wid * NCH, NCH)], buf)

    @pl.when(wid == 30)
    def _boundary():
        pltpu.sync_copy(tbl_hbm.at[row, pl.ds(30 * NCH, B30)],
                        buf.at[pl.ds(0, B30)])
        pltpu.sync_copy(loops_hbm.at[row, pl.ds(0, NCH - B30)],
                        buf.at[pl.ds(B30, NCH - B30)])

    @pl.when(wid == 31)
    def _all_loops():
        pltpu.sync_copy(loops_hbm.at[row, pl.ds(NCH - B30, NCH)], buf)


@functools.partial(
    pl.kernel,
    out_type=jax.ShapeDtypeStruct((2, 2, N_NODES, D), jnp.float32),
    mesh=_mesh,
    scratch_types=[
        pltpu.VMEM((NCH, CH), jnp.int32),
        pltpu.VMEM((NCH, CH), jnp.int32),
        pltpu.VMEM((CH, D), jnp.float32),
        pltpu.SemaphoreType.DMA,
        pltpu.VMEM_SHARED((NP, D), jnp.float32),
        pltpu.VMEM_SHARED((NP, D), jnp.float32),
    ],
    compiler_params=_sc_params,
)
def _deg_kernel(eip_hbm, ein_hbm, loops_hbm, z_hbm, ones_hbm, out_hbm,
                dpv, dnv, ones_v, sem_s, accp, accn):
    cid = lax.axis_index("c")
    sid = lax.axis_index("s")
    wid = cid * 16 + sid
    st = NP // 16
    stg = N_NODES // 16
    pltpu.sync_copy(z_hbm.at[pl.ds(sid * st, st)], accp.at[pl.ds(sid * st, st)])
    pltpu.sync_copy(z_hbm.at[pl.ds(sid * st, st)], accn.at[pl.ds(sid * st, st)])
    pltpu.sync_copy(ones_hbm, ones_v)
    _load_idx(eip_hbm, loops_hbm, 1, dpv, wid)
    _load_idx(ein_hbm, loops_hbm, 1, dnv, wid)
    plsc.subcore_barrier()

    def one_sign(dv, acc):
        def fire(j, carry):
            pltpu.async_copy(ones_v, acc.at[dv.at[j]], sem_s, add=True)
            return carry
        lax.fori_loop(0, NCH, fire, 0)

        def drain(j, carry):
            pltpu.make_async_copy(ones_v, acc.at[dv.at[0]], sem_s).wait()
            return carry
        lax.fori_loop(0, NCH, drain, 0)

    one_sign(dpv, accp)
    one_sign(dnv, accn)
    plsc.subcore_barrier()
    pltpu.sync_copy(accp.at[pl.ds(sid * stg, stg)],
                    out_hbm.at[cid, 0, pl.ds(sid * stg, stg)])
    pltpu.sync_copy(accn.at[pl.ds(sid * stg, stg)],
                    out_hbm.at[cid, 1, pl.ds(sid * stg, stg)])


@functools.partial(
    pl.kernel,
    out_type=[jax.ShapeDtypeStruct((2, N_NODES, D), jnp.float32),
              jax.ShapeDtypeStruct((2, N_NODES, D), jnp.float32)],
    mesh=_mesh,
    scratch_types=[
        pltpu.VMEM((NCH, CH), jnp.int32),
        pltpu.VMEM((NCH, CH), jnp.int32),
        pltpu.VMEM((NCH, CH), jnp.int32),
        pltpu.VMEM((NCH, CH), jnp.int32),
        pltpu.VMEM((2, KG, CH, D), jnp.float32),
        pltpu.SemaphoreType.DMA,
        pltpu.SemaphoreType.DMA,
        pltpu.VMEM_SHARED((NP, D), jnp.float32),
        pltpu.VMEM_SHARED((NP, D), jnp.float32),
        pltpu.VMEM_SHARED((N_NODES, D), jnp.float32),
        pltpu.VMEM_SHARED((N_NODES, D), jnp.float32),
    ],
    compiler_params=_sc_params,
)
def _conv_kernel(gp_hbm, gn_hbm, z_hbm, eip_hbm, ein_hbm, loops_hbm,
                 yp_hbm, yn_hbm, spv, dpv, snv, dnv, rows, sem_g, sem_s,
                 accp, accn, gsp, gsn):
    cid = lax.axis_index("c")
    sid = lax.axis_index("s")
    wid = cid * 16 + sid
    st = NP // 16
    stg = N_NODES // 16
    pltpu.sync_copy(z_hbm.at[pl.ds(sid * st, st)], accp.at[pl.ds(sid * st, st)])
    pltpu.sync_copy(z_hbm.at[pl.ds(sid * st, st)], accn.at[pl.ds(sid * st, st)])
    pltpu.sync_copy(gp_hbm.at[pl.ds(sid * stg, stg)],
                    gsp.at[pl.ds(sid * stg, stg)])
    pltpu.sync_copy(gn_hbm.at[pl.ds(sid * stg, stg)],
                    gsn.at[pl.ds(sid * stg, stg)])
    _load_idx(eip_hbm, loops_hbm, 0, spv, wid)
    _load_idx(eip_hbm, loops_hbm, 1, dpv, wid)
    _load_idx(ein_hbm, loops_hbm, 0, snv, wid)
    _load_idx(ein_hbm, loops_hbm, 1, dnv, wid)
    plsc.subcore_barrier()

    NG = NCH // KG

    def one_sign(g_tbl, sv, dv, acc):
        for k in range(KG):
            pltpu.async_copy(g_tbl.at[sv.at[k]], rows.at[0, k], sem_g)

        def group(t, carry):
            par = lax.rem(t, 2)
            nxt = 1 - par
            base = t * KG

            @pl.when(t + 1 < NG)
            def _fire_next():
                @pl.when(t >= 1)
                def _drain_prev_scatters():
                    for k in range(KG):
                        pltpu.make_async_copy(
                            rows.at[nxt, k],
                            acc.at[dv.at[base - KG + k]], sem_s).wait()
                for k in range(KG):
                    pltpu.async_copy(g_tbl.at[sv.at[base + KG + k]],
                                     rows.at[nxt, k], sem_g)

            for k in range(KG):
                pltpu.make_async_copy(g_tbl.at[sv.at[base + k]],
                                      rows.at[par, k], sem_g).wait()
                pltpu.async_copy(rows.at[par, k], acc.at[dv.at[base + k]],
                                 sem_s, add=True)
            return carry

        lax.fori_loop(0, NG, group, 0)
        for k in range(2 * KG):
            pltpu.make_async_copy(rows.at[0, 0], acc.at[dv.at[0]],
                                  sem_s).wait()

    one_sign(gsp, spv, dpv, accp)
    one_sign(gsn, snv, dnv, accn)
    plsc.subcore_barrier()
    pltpu.sync_copy(accp.at[pl.ds(sid * stg, stg)],
                    yp_hbm.at[cid, pl.ds(sid * stg, stg)])
    pltpu.sync_copy(accn.at[pl.ds(sid * stg, stg)],
                    yn_hbm.at[cid, pl.ds(sid * stg, stg)])


_GRID = 5
_BR = N_NODES // _GRID


def _dis(deg):
    return jnp.where(deg > 0.0, lax.rsqrt(deg), 0.0)


def _dense1_body(x_ref, w1p_ref, w1n_ref, deg_ref,
                 gp_ref, gn_ref, dp_ref, dn_ref):
    deg = deg_ref[...]
    disp = _dis(deg[0, 0] + deg[1, 0])
    disn = _dis(deg[0, 1] + deg[1, 1])
    hp = jnp.dot(x_ref[...], w1p_ref[...], preferred_element_type=jnp.float32)
    hn = jnp.dot(x_ref[...], w1n_ref[...], preferred_element_type=jnp.float32)
    gp_ref[...] = hp * disp
    gn_ref[...] = hn * disn
    dp_ref[...] = disp
    dn_ref[...] = disn


_dense1 = pl.pallas_call(
    _dense1_body,
    grid=(_GRID,),
    in_specs=[
        pl.BlockSpec((_BR, F), lambda i: (i, 0)),
        pl.BlockSpec((F, D), lambda i: (0, 0)),
        pl.BlockSpec((F, D), lambda i: (0, 0)),
        pl.BlockSpec((2, 2, _BR, D), lambda i: (0, 0, i, 0)),
    ],
    out_specs=[pl.BlockSpec((_BR, D), lambda i: (i, 0))] * 4,
    out_shape=[jax.ShapeDtypeStruct((N_NODES, D), jnp.float32)] * 4,
    compiler_params=_tc_params,
)


def _dense2_body(ypp_ref, ynp_ref, dp_ref, dn_ref,
                 b1p_ref, b1n_ref, w2p_ref, w2n_ref,
                 gp2_ref, gn2_ref):
    disp = dp_ref[...]
    disn = dn_ref[...]
    yp = ypp_ref[0] + ypp_ref[1]
    yn = ynp_ref[0] + ynp_ref[1]
    ap = jnp.maximum(disp * yp + b1p_ref[...], 0.0)
    an = jnp.maximum(disn * yn + b1n_ref[...], 0.0)
    h = ap - an
    hp2 = jnp.dot(h, w2p_ref[...], preferred_element_type=jnp.float32)
    hn2 = jnp.dot(h, w2n_ref[...], preferred_element_type=jnp.float32)
    gp2_ref[...] = hp2 * disp
    gn2_ref[...] = hn2 * disn


_dense2 = pl.pallas_call(
    _dense2_body,
    grid=(_GRID,),
    in_specs=[
        pl.BlockSpec((2, _BR, D), lambda i: (0, i, 0)),
        pl.BlockSpec((2, _BR, D), lambda i: (0, i, 0)),
        pl.BlockSpec((_BR, D), lambda i: (i, 0)),
        pl.BlockSpec((_BR, D), lambda i: (i, 0)),
        pl.BlockSpec((1, D), lambda i: (0, 0)),
        pl.BlockSpec((1, D), lambda i: (0, 0)),
        pl.BlockSpec((D, D), lambda i: (0, 0)),
        pl.BlockSpec((D, D), lambda i: (0, 0)),
    ],
    out_specs=[pl.BlockSpec((_BR, D), lambda i: (i, 0))] * 2,
    out_shape=[jax.ShapeDtypeStruct((N_NODES, D), jnp.float32)] * 2,
    compiler_params=_tc_params,
)


def _dense3_body(ypp_ref, ynp_ref, dp_ref, dn_ref,
                 b2p_ref, b2n_ref, out_ref):
    disp = dp_ref[...]
    disn = dn_ref[...]
    yp = ypp_ref[0] + ypp_ref[1]
    yn = ynp_ref[0] + ynp_ref[1]
    op = jnp.maximum(disp * yp + b2p_ref[...], 0.0)
    on = jnp.maximum(disn * yn + b2n_ref[...], 0.0)
    o = op - on
    m = jnp.max(o, axis=1, keepdims=True)
    lse = jnp.log(jnp.sum(jnp.exp(o - m), axis=1, keepdims=True)) + m
    out_ref[...] = o - lse


_dense3 = pl.pallas_call(
    _dense3_body,
    grid=(_GRID,),
    in_specs=[
        pl.BlockSpec((2, _BR, D), lambda i: (0, i, 0)),
        pl.BlockSpec((2, _BR, D), lambda i: (0, i, 0)),
        pl.BlockSpec((_BR, D), lambda i: (i, 0)),
        pl.BlockSpec((_BR, D), lambda i: (i, 0)),
        pl.BlockSpec((1, D), lambda i: (0, 0)),
        pl.BlockSpec((1, D), lambda i: (0, 0)),
    ],
    out_specs=pl.BlockSpec((_BR, D), lambda i: (i, 0)),
    out_shape=jax.ShapeDtypeStruct((N_NODES, D), jnp.float32),
    compiler_params=_tc_params,
)


def kernel(x, edge_index_pos, edge_index_neg,
           W1p, b1p, W1n, b1n, W2p, b2p, W2n, b2n):
    eip_r = edge_index_pos.astype(jnp.int32).reshape(2, EC, CH)
    ein_r = edge_index_neg.astype(jnp.int32).reshape(2, EC, CH)
    loop = jnp.arange(N_NODES, dtype=jnp.int32)
    loops_r = jnp.stack([
        jnp.concatenate([loop, jnp.zeros((LC * CH - N_NODES,), jnp.int32)]),
        jnp.concatenate([loop, jnp.full((LC * CH - N_NODES,), NP - 1,
                                        jnp.int32)]),
    ]).reshape(2, LC, CH)
    zeros_tbl = jnp.zeros((NP, D), jnp.float32)
    ones_tbl = jnp.ones((CH, D), jnp.float32)

    deg = _deg_kernel(eip_r, ein_r, loops_r, zeros_tbl, ones_tbl)
    gp, gn, disp, disn = _dense1(x, W1p, W1n, deg)
    ypp, ynp = _conv_kernel(gp, gn, zeros_tbl, eip_r, ein_r, loops_r)
    gp2, gn2 = _dense2(ypp, ynp, disp, disn,
                       b1p.reshape(1, D), b1n.reshape(1, D), W2p, W2n)
    ypp2, ynp2 = _conv_kernel(gp2, gn2, zeros_tbl, eip_r, ein_r, loops_r)
    o = _dense3(ypp2, ynp2, disp, disn,
                b2p.reshape(1, D), b2n.reshape(1, D))
    return o
